# Initial kernel scaffold; baseline (speedup 1.0000x reference)
import functools

import jax
import jax.numpy as jnp
from jax import lax
from jax.experimental import pallas as pl
from jax.experimental.pallas import tpu as pltpu
from jax.experimental.pallas import tpu_sc as plsc

N = 10000
E = 320000
D_IN = 128
D_HID = 16
D_OUT = 128

NC = 2
NS = 16
NW = NC * NS
EPW = E // NW
B = 80
CH = EPW // B
N_PAD = 10240
RPS = N_PAD // NS

_mesh = functools.partial(
    pl.kernel,
    mesh=plsc.VectorSubcoreMesh(core_axis_name="c", subcore_axis_name="s"),
)


def _worker_id():
    return lax.axis_index("s") * NC + lax.axis_index("c")


@_mesh(
    out_type=jax.ShapeDtypeStruct((NC, N_PAD, D_HID), jnp.float32),
    scratch_types=[
        pltpu.VMEM((CH, B), jnp.int32),
        pltpu.VMEM((B, D_HID), jnp.float32),
        pltpu.VMEM_SHARED((N_PAD, D_HID), jnp.float32),
    ],
)
def _sc_degree(dst3, zeros_hbm, ones_hbm, out, dst_v, ones_v, acc):
    c = lax.axis_index("c")
    sid = lax.axis_index("s")
    wid = _worker_id()
    r0 = sid * RPS
    pltpu.sync_copy(zeros_hbm.at[pl.ds(r0, RPS)], acc.at[pl.ds(r0, RPS)])
    pltpu.sync_copy(ones_hbm, ones_v)
    pltpu.sync_copy(dst3.at[wid], dst_v)
    plsc.subcore_barrier()

    def body(j, carry):
        pltpu.sync_copy(ones_v, acc.at[dst_v.at[j]], add=True)
        return carry

    lax.fori_loop(0, CH, body, 0)
    plsc.subcore_barrier()
    pltpu.sync_copy(acc.at[pl.ds(r0, RPS)], out.at[c, pl.ds(r0, RPS)])


@_mesh(
    out_type=jax.ShapeDtypeStruct((NC, N_PAD, D_HID), jnp.float32),
    scratch_types=[
        pltpu.VMEM((CH, B), jnp.int32),
        pltpu.VMEM((CH, B), jnp.int32),
        pltpu.VMEM((B, D_HID), jnp.float32),
        pltpu.VMEM((B, D_HID), jnp.float32),
        pltpu.VMEM_SHARED((N_PAD, D_HID), jnp.float32),
        pltpu.SemaphoreType.DMA,
        pltpu.SemaphoreType.DMA,
    ],
)
def _sc_aggregate(src3, dst3, g_hbm, zeros_hbm, out,
                  src_v, dst_v, rows0, rows1, acc, sem0, sem1):
    c = lax.axis_index("c")
    sid = lax.axis_index("s")
    wid = _worker_id()
    r0 = sid * RPS
    pltpu.sync_copy(zeros_hbm.at[pl.ds(r0, RPS)], acc.at[pl.ds(r0, RPS)])
    pltpu.sync_copy(src3.at[wid], src_v)
    pltpu.sync_copy(dst3.at[wid], dst_v)
    plsc.subcore_barrier()

    pltpu.async_copy(g_hbm.at[src_v.at[0]], rows0, sem0)

    def body(j, carry):
        @pl.when(j + 1 < CH)
        def _():
            @pl.when(lax.rem(j, 2) == 0)
            def _():
                pltpu.async_copy(g_hbm.at[src_v.at[j + 1]], rows1, sem1)

            @pl.when(lax.rem(j, 2) == 1)
            def _():
                pltpu.async_copy(g_hbm.at[src_v.at[j + 1]], rows0, sem0)

        @pl.when(lax.rem(j, 2) == 0)
        def _():
            pltpu.make_async_copy(g_hbm.at[src_v.at[j]], rows0, sem0).wait()
            pltpu.sync_copy(rows0, acc.at[dst_v.at[j]], add=True)

        @pl.when(lax.rem(j, 2) == 1)
        def _():
            pltpu.make_async_copy(g_hbm.at[src_v.at[j]], rows1, sem1).wait()
            pltpu.sync_copy(rows1, acc.at[dst_v.at[j]], add=True)

        return carry

    lax.fori_loop(0, CH, body, 0)
    plsc.subcore_barrier()
    pltpu.sync_copy(acc.at[pl.ds(r0, RPS)], out.at[c, pl.ds(r0, RPS)])


def _tc_a_body(x_ref, w1_ref, degacc_ref, g1_ref, dinv_ref):
    deg = 1.0 + degacc_ref[0, :, 0:1] + degacc_ref[1, :, 0:1]
    dinv = lax.rsqrt(deg)
    dinv_ref[...] = dinv
    h1 = jnp.dot(x_ref[...], w1_ref[...], preferred_element_type=jnp.float32)
    g1_ref[...] = dinv[:N] * h1


def _tc_b_body(s1_ref, g1_ref, dinv_ref, b1_ref, g2_ref):
    dinv = dinv_ref[:N]
    s = s1_ref[0, :N] + s1_ref[1, :N] + g1_ref[...]
    h = jnp.maximum(dinv * s + b1_ref[...], 0.0)
    g2_ref[...] = dinv * h


def _tc_c_body(s2_ref, g2_ref, dinv_ref, w2_ref, b2_ref, out_ref):
    a = dinv_ref[:N] * (s2_ref[0, :N] + s2_ref[1, :N] + g2_ref[...])
    out_ref[...] = (
        jnp.dot(a, w2_ref[...], preferred_element_type=jnp.float32)
        + b2_ref[...]
    )


def kernel(x, edge_index, W1, b1, W2, b2):
    src3 = edge_index[0].reshape(NW, CH, B)
    dst3 = edge_index[1].reshape(NW, CH, B)
    zeros = jnp.zeros((N_PAD, D_HID), jnp.float32)
    ones = jnp.ones((B, D_HID), jnp.float32)

    degacc = _sc_degree(dst3, zeros, ones)

    g1, dinv = pl.pallas_call(
        _tc_a_body,
        out_shape=(
            jax.ShapeDtypeStruct((N, D_HID), jnp.float32),
            jax.ShapeDtypeStruct((N_PAD, 1), jnp.float32),
        ),
    )(x, W1, degacc)

    s1 = _sc_aggregate(src3, dst3, g1, zeros)

    g2 = pl.pallas_call(
        _tc_b_body,
        out_shape=jax.ShapeDtypeStruct((N, D_HID), jnp.float32),
    )(s1, g1, dinv, b1.reshape(1, D_HID))

    s2 = _sc_aggregate(src3, dst3, g2, zeros)

    out = pl.pallas_call(
        _tc_c_body,
        out_shape=jax.ShapeDtypeStruct((N, D_OUT), jnp.float32),
    )(s2, g2, dinv, W2, b2.reshape(1, D_OUT))

    return out


# speedup vs baseline: 42.4467x; 42.4467x over previous
import functools

import jax
import jax.numpy as jnp
from jax import lax
from jax.experimental import pallas as pl
from jax.experimental.pallas import tpu as pltpu
from jax.experimental.pallas import tpu_sc as plsc

N = 10000
E = 320000
D_IN = 128
D_HID = 16
D_OUT = 128

NC = 2
NS = 16
NW = NC * NS
EPW = E // NW
B = 80
CH = EPW // B
N_PAD = 10240
RPS = N_PAD // NS

_mesh = functools.partial(
    pl.kernel,
    mesh=plsc.VectorSubcoreMesh(core_axis_name="c", subcore_axis_name="s"),
    compiler_params=pltpu.CompilerParams(use_tc_tiling_on_sc=False),
)


def _worker_id():
    return lax.axis_index("s") * NC + lax.axis_index("c")


@_mesh(
    out_type=jax.ShapeDtypeStruct((NC, N_PAD, D_HID), jnp.float32),
    scratch_types=[
        pltpu.VMEM((CH, B), jnp.int32),
        pltpu.VMEM((B, D_HID), jnp.float32),
        pltpu.VMEM_SHARED((N_PAD, D_HID), jnp.float32),
    ],
)
def _sc_degree(dst3, zeros_hbm, ones_hbm, out, dst_v, ones_v, acc):
    c = lax.axis_index("c")
    sid = lax.axis_index("s")
    wid = _worker_id()
    r0 = sid * RPS
    pltpu.sync_copy(zeros_hbm.at[pl.ds(r0, RPS)], acc.at[pl.ds(r0, RPS)])
    pltpu.sync_copy(ones_hbm, ones_v)
    pltpu.sync_copy(dst3.at[wid], dst_v)
    plsc.subcore_barrier()

    def body(j, carry):
        pltpu.sync_copy(ones_v, acc.at[dst_v.at[j]], add=True)
        return carry

    lax.fori_loop(0, CH, body, 0)
    plsc.subcore_barrier()
    pltpu.sync_copy(acc.at[pl.ds(r0, RPS)], out.at[c, pl.ds(r0, RPS)])


@_mesh(
    out_type=jax.ShapeDtypeStruct((NC, N_PAD, D_HID), jnp.float32),
    scratch_types=[
        pltpu.VMEM((CH, B), jnp.int32),
        pltpu.VMEM((CH, B), jnp.int32),
        pltpu.VMEM((B, D_HID), jnp.float32),
        pltpu.VMEM((B, D_HID), jnp.float32),
        pltpu.VMEM_SHARED((N_PAD, D_HID), jnp.float32),
        pltpu.SemaphoreType.DMA,
        pltpu.SemaphoreType.DMA,
    ],
)
def _sc_aggregate(src3, dst3, g_hbm, zeros_hbm, out,
                  src_v, dst_v, rows0, rows1, acc, sem0, sem1):
    c = lax.axis_index("c")
    sid = lax.axis_index("s")
    wid = _worker_id()
    r0 = sid * RPS
    pltpu.sync_copy(zeros_hbm.at[pl.ds(r0, RPS)], acc.at[pl.ds(r0, RPS)])
    pltpu.sync_copy(src3.at[wid], src_v)
    pltpu.sync_copy(dst3.at[wid], dst_v)
    plsc.subcore_barrier()

    pltpu.async_copy(g_hbm.at[src_v.at[0]], rows0, sem0)

    def body(j, carry):
        @pl.when(j + 1 < CH)
        def _():
            @pl.when(lax.rem(j, 2) == 0)
            def _():
                pltpu.async_copy(g_hbm.at[src_v.at[j + 1]], rows1, sem1)

            @pl.when(lax.rem(j, 2) == 1)
            def _():
                pltpu.async_copy(g_hbm.at[src_v.at[j + 1]], rows0, sem0)

        @pl.when(lax.rem(j, 2) == 0)
        def _():
            pltpu.make_async_copy(g_hbm.at[src_v.at[j]], rows0, sem0).wait()
            pltpu.sync_copy(rows0, acc.at[dst_v.at[j]], add=True)

        @pl.when(lax.rem(j, 2) == 1)
        def _():
            pltpu.make_async_copy(g_hbm.at[src_v.at[j]], rows1, sem1).wait()
            pltpu.sync_copy(rows1, acc.at[dst_v.at[j]], add=True)

        return carry

    lax.fori_loop(0, CH, body, 0)
    plsc.subcore_barrier()
    pltpu.sync_copy(acc.at[pl.ds(r0, RPS)], out.at[c, pl.ds(r0, RPS)])


def _tc_a_body(x_ref, w1_ref, degacc_ref, g1_ref, dinv_ref):
    deg = 1.0 + degacc_ref[0, :, 0:1] + degacc_ref[1, :, 0:1]
    dinv = lax.rsqrt(deg)
    dinv_ref[...] = dinv
    h1 = jnp.dot(x_ref[...], w1_ref[...], preferred_element_type=jnp.float32)
    g1_ref[...] = dinv[:N] * h1


def _tc_b_body(s1_ref, g1_ref, dinv_ref, b1_ref, g2_ref):
    dinv = dinv_ref[:N]
    s = s1_ref[0, :N] + s1_ref[1, :N] + g1_ref[...]
    h = jnp.maximum(dinv * s + b1_ref[...], 0.0)
    g2_ref[...] = dinv * h


def _tc_c_body(s2_ref, g2_ref, dinv_ref, w2_ref, b2_ref, out_ref):
    a = dinv_ref[:N] * (s2_ref[0, :N] + s2_ref[1, :N] + g2_ref[...])
    out_ref[...] = (
        jnp.dot(a, w2_ref[...], preferred_element_type=jnp.float32)
        + b2_ref[...]
    )


def kernel(x, edge_index, W1, b1, W2, b2):
    src3 = edge_index[0].reshape(NW, CH, B)
    dst3 = edge_index[1].reshape(NW, CH, B)
    zeros = jnp.zeros((N_PAD, D_HID), jnp.float32)
    ones = jnp.ones((B, D_HID), jnp.float32)

    degacc = _sc_degree(dst3, zeros, ones)

    g1, dinv = pl.pallas_call(
        _tc_a_body,
        out_shape=(
            jax.ShapeDtypeStruct((N, D_HID), jnp.float32),
            jax.ShapeDtypeStruct((N_PAD, 1), jnp.float32),
        ),
    )(x, W1, degacc)

    s1 = _sc_aggregate(src3, dst3, g1, zeros)

    g2 = pl.pallas_call(
        _tc_b_body,
        out_shape=jax.ShapeDtypeStruct((N, D_HID), jnp.float32),
    )(s1, g1, dinv, b1.reshape(1, D_HID))

    s2 = _sc_aggregate(src3, dst3, g2, zeros)

    out = pl.pallas_call(
        _tc_c_body,
        out_shape=jax.ShapeDtypeStruct((N, D_OUT), jnp.float32),
    )(s2, g2, dinv, W2, b2.reshape(1, D_OUT))

    return out
